# baseline (speedup 1.0000x reference)
import functools

import jax
import jax.numpy as jnp
from jax import lax
from jax.experimental import pallas as pl
from jax.experimental.pallas import tpu as pltpu
from jax.experimental.pallas import tpu_sc as plsc

SEQ, B, D = 2048, 16, 768
POOL, L2, TOP_K = 4096, 8, 8
HALF = L2 // 2
ROW = HALF * D
NIDX = B * TOP_K
EPS = 1e-12

_SEQ_BLK = 128


def _mean_norm_body(x_ref, o_ref):
    i = pl.program_id(0)
    part = jnp.sum(x_ref[...], axis=0)

    @pl.when(i == 0)
    def _():
        o_ref[...] = part

    @pl.when(i > 0)
    def _():
        o_ref[...] += part

    @pl.when(i == pl.num_programs(0) - 1)
    def _():
        m = o_ref[...] * (1.0 / SEQ)
        ssq = jnp.sum(m * m, axis=1, keepdims=True)
        o_ref[...] = m * lax.rsqrt(jnp.maximum(ssq, EPS))


def _mean_norm(x3d):
    return pl.pallas_call(
        _mean_norm_body,
        grid=(SEQ // _SEQ_BLK,),
        in_specs=[pl.BlockSpec((_SEQ_BLK, B, D), lambda i: (i, 0, 0))],
        out_specs=pl.BlockSpec((B, D), lambda i: (0, 0)),
        out_shape=jax.ShapeDtypeStruct((B, D), jnp.float32),
        compiler_params=pltpu.CompilerParams(
            dimension_semantics=("arbitrary",)),
    )(x3d)


_POOL_BLK = 1024


def _pk_sim_body(xn_ref, pk_ref, pkraw_ref, pkn_ref, sim_ref, bkn_ref, rs_ref):
    i = pl.program_id(0)
    pk = pk_ref[...]
    ssq = jnp.sum(pk * pk, axis=1, keepdims=True)
    pkn = pk * lax.rsqrt(jnp.maximum(ssq, EPS))
    pkn_ref[...] = pkn
    xn = xn_ref[...]
    sim_ref[...] = lax.dot_general(
        xn, pkn, (((1,), (1,)), ((), ())),
        preferred_element_type=jnp.float32)

    @pl.when(i == pl.num_programs(0) - 1)
    def _():
        g = pkraw_ref[...]
        gssq = jnp.sum(g * g, axis=1, keepdims=True)
        bkn = g * lax.rsqrt(jnp.maximum(gssq, EPS))
        bkn_ref[...] = bkn
        xr = jnp.broadcast_to(xn[:, None, :], (B, TOP_K, D)).reshape(NIDX, D)
        rs_ref[0, 0] = jnp.sum(bkn * xr) * (1.0 / (B * TOP_K))


def _pk_sim(x_norm, prompt_key, pkraw):
    return pl.pallas_call(
        _pk_sim_body,
        grid=(POOL // _POOL_BLK,),
        in_specs=[
            pl.BlockSpec((B, D), lambda i: (0, 0)),
            pl.BlockSpec((_POOL_BLK, D), lambda i: (i, 0)),
            pl.BlockSpec((NIDX, D), lambda i: (0, 0)),
        ],
        out_specs=[
            pl.BlockSpec((_POOL_BLK, D), lambda i: (i, 0)),
            pl.BlockSpec((B, _POOL_BLK), lambda i: (0, i)),
            pl.BlockSpec((NIDX, D), lambda i: (0, 0)),
            pl.BlockSpec(memory_space=pltpu.SMEM),
        ],
        out_shape=[
            jax.ShapeDtypeStruct((POOL, D), jnp.float32),
            jax.ShapeDtypeStruct((B, POOL), jnp.float32),
            jax.ShapeDtypeStruct((NIDX, D), jnp.float32),
            jax.ShapeDtypeStruct((1, 1), jnp.float32),
        ],
    )(x_norm, prompt_key, pkraw)


_LT = D // 128


def _sc_gather_body(tab_ref, idx_ref, pk_ref, key_ref, val_ref, pkraw_ref,
                    idx_v, rows_v, kbuf, vbuf, pidx_v, pkrows_v, sem):
    wid = lax.axis_index("s") * 2 + lax.axis_index("c")

    @pl.when(wid < 16)
    def _():
        b = wid
        pltpu.sync_copy(idx_ref.at[b], idx_v)
        pltpu.async_copy(tab_ref.at[idx_v], rows_v, sem).wait()
        for k in range(TOP_K):
            for h in range(HALF):

                def piece(m, carry, k=k, h=h):
                    kbuf[HALF * k + h, pl.ds(16 * m, 16)] = (
                        rows_v[k, h, pl.ds(16 * m, 16)])
                    vbuf[HALF * k + h, pl.ds(16 * m, 16)] = (
                        rows_v[k, HALF + h, pl.ds(16 * m, 16)])
                    return carry

                lax.fori_loop(0, D // 16, piece, 0)
        o0 = pl.multiple_of(32 * b, 32)
        pltpu.sync_copy(kbuf, key_ref.at[pl.ds(o0, 32)])
        pltpu.sync_copy(vbuf, val_ref.at[pl.ds(o0, 32)])

    @pl.when(wid >= 16)
    def _():
        w = wid - 16
        pltpu.sync_copy(idx_ref.at[w], pidx_v)
        pltpu.async_copy(pk_ref.at[pidx_v], pkrows_v, sem).wait()
        pltpu.sync_copy(pkrows_v, pkraw_ref.at[pl.ds(pl.multiple_of(8 * w, 8), 8)])


@functools.cache
def _sc_gather_fn():
    mesh = plsc.VectorSubcoreMesh(core_axis_name="c", subcore_axis_name="s")
    return pl.kernel(
        _sc_gather_body,
        out_type=(
            jax.ShapeDtypeStruct((B * TOP_K * HALF, D), jnp.float32),
            jax.ShapeDtypeStruct((B * TOP_K * HALF, D), jnp.float32),
            jax.ShapeDtypeStruct((NIDX, D), jnp.float32),
        ),
        mesh=mesh,
        scratch_types=(
            pltpu.VMEM((8,), jnp.int32),
            pltpu.VMEM((8, L2, D), jnp.float32),
            pltpu.VMEM((TOP_K * HALF, D), jnp.float32),
            pltpu.VMEM((TOP_K * HALF, D), jnp.float32),
            pltpu.VMEM((8,), jnp.int32),
            pltpu.VMEM((8, D), jnp.float32),
            pltpu.SemaphoreType.DMA,
        ),
    )


def _scs_gather_body(tab_ref, idx_ref, pk_ref, key_ref, val_ref, pkraw_ref,
                     idx_sm, sem):
    c = lax.axis_index("c")
    pltpu.sync_copy(idx_ref, idx_sm)
    handles = []
    for j in range(64):
        b = 8 * c + j // 8
        k = j % 8
        i = idx_sm[b, k]
        o = pl.multiple_of(32 * b, 32) + 4 * k
        handles.append(pltpu.async_copy(
            tab_ref.at[i, pl.ds(0, HALF)], key_ref.at[pl.ds(o, HALF)], sem))
        handles.append(pltpu.async_copy(
            tab_ref.at[i, pl.ds(HALF, HALF)], val_ref.at[pl.ds(o, HALF)], sem))
        handles.append(pltpu.async_copy(
            pk_ref.at[i], pkraw_ref.at[8 * b + k], sem))
    for h in handles:
        h.wait()


@functools.cache
def _scs_gather_fn():
    mesh = plsc.ScalarSubcoreMesh(axis_name="c", num_cores=2)
    return pl.kernel(
        _scs_gather_body,
        out_type=(
            jax.ShapeDtypeStruct((B * TOP_K * HALF, D), jnp.float32),
            jax.ShapeDtypeStruct((B * TOP_K * HALF, D), jnp.float32),
            jax.ShapeDtypeStruct((NIDX, D), jnp.float32),
        ),
        mesh=mesh,
        scratch_types=(
            pltpu.SMEM((B, TOP_K), jnp.int32),
            pltpu.SemaphoreType.DMA,
        ),
    )


def kernel(x_embed, prompt_mask, cls_features, prompt, prompt_key):
    del cls_features
    key_rows, val_rows, pkraw = _scs_gather_fn()(prompt, prompt_mask, prompt_key)

    x_norm = _mean_norm(x_embed)
    pk_norm, similarity, bkn, rs = _pk_sim(x_norm, prompt_key, pkraw)

    key_prompt = key_rows.reshape(B, TOP_K * HALF, D)
    value_prompt = val_rows.reshape(B, TOP_K * HALF, D)
    batched_key_norm = bkn.reshape(B, TOP_K, D)
    reduce_sim = rs[0, 0]
    return (similarity, prompt_mask, key_prompt, value_prompt,
            batched_key_norm, pk_norm, x_norm, reduce_sim)


# speedup vs baseline: 2.2546x; 2.2546x over previous
import functools

import jax
import jax.numpy as jnp
from jax import lax
from jax.experimental import pallas as pl
from jax.experimental.pallas import tpu as pltpu
from jax.experimental.pallas import tpu_sc as plsc

SEQ, B, D = 2048, 16, 768
POOL, L2, TOP_K = 4096, 8, 8
HALF = L2 // 2
ROW = HALF * D
NIDX = B * TOP_K
EPS = 1e-12

_SEQ_BLK = 128


def _mean_norm_body(x_ref, o_ref):
    i = pl.program_id(0)
    part = jnp.sum(x_ref[...], axis=0)

    @pl.when(i == 0)
    def _():
        o_ref[...] = part

    @pl.when(i > 0)
    def _():
        o_ref[...] += part

    @pl.when(i == pl.num_programs(0) - 1)
    def _():
        m = o_ref[...] * (1.0 / SEQ)
        ssq = jnp.sum(m * m, axis=1, keepdims=True)
        o_ref[...] = m * lax.rsqrt(jnp.maximum(ssq, EPS))


def _mean_norm(x3d):
    return pl.pallas_call(
        _mean_norm_body,
        grid=(SEQ // _SEQ_BLK,),
        in_specs=[pl.BlockSpec((_SEQ_BLK, B, D), lambda i: (i, 0, 0))],
        out_specs=pl.BlockSpec((B, D), lambda i: (0, 0)),
        out_shape=jax.ShapeDtypeStruct((B, D), jnp.float32),
        compiler_params=pltpu.CompilerParams(
            dimension_semantics=("arbitrary",)),
    )(x3d)


_POOL_BLK = 1024


def _pk_sim_body(xn_ref, pk_ref, pkraw_ref, pkn_ref, sim_ref, bkn_ref, rs_ref):
    i = pl.program_id(0)
    pk = pk_ref[...]
    ssq = jnp.sum(pk * pk, axis=1, keepdims=True)
    pkn = pk * lax.rsqrt(jnp.maximum(ssq, EPS))
    pkn_ref[...] = pkn
    xn = xn_ref[...]
    sim_ref[...] = lax.dot_general(
        xn, pkn, (((1,), (1,)), ((), ())),
        preferred_element_type=jnp.float32)

    @pl.when(i == pl.num_programs(0) - 1)
    def _():
        g = pkraw_ref[...]
        gssq = jnp.sum(g * g, axis=1, keepdims=True)
        bkn = g * lax.rsqrt(jnp.maximum(gssq, EPS))
        bkn_ref[...] = bkn
        xr = jnp.broadcast_to(xn[:, None, :], (B, TOP_K, D)).reshape(NIDX, D)
        rs_ref[0, 0] = jnp.sum(bkn * xr) * (1.0 / (B * TOP_K))


def _pk_sim(x_norm, prompt_key, pkraw):
    return pl.pallas_call(
        _pk_sim_body,
        grid=(POOL // _POOL_BLK,),
        in_specs=[
            pl.BlockSpec((B, D), lambda i: (0, 0)),
            pl.BlockSpec((_POOL_BLK, D), lambda i: (i, 0)),
            pl.BlockSpec((NIDX, D), lambda i: (0, 0)),
        ],
        out_specs=[
            pl.BlockSpec((_POOL_BLK, D), lambda i: (i, 0)),
            pl.BlockSpec((B, _POOL_BLK), lambda i: (0, i)),
            pl.BlockSpec((NIDX, D), lambda i: (0, 0)),
            pl.BlockSpec(memory_space=pltpu.SMEM),
        ],
        out_shape=[
            jax.ShapeDtypeStruct((POOL, D), jnp.float32),
            jax.ShapeDtypeStruct((B, POOL), jnp.float32),
            jax.ShapeDtypeStruct((NIDX, D), jnp.float32),
            jax.ShapeDtypeStruct((1, 1), jnp.float32),
        ],
    )(x_norm, prompt_key, pkraw)


_LT = D // 128


def _sc_gather_body(tab_ref, idx_ref, pk_ref, key_ref, val_ref, pkraw_ref,
                    idx_v, rows_v, pidx_v, pkrows_v, sem):
    wid = lax.axis_index("s") * 2 + lax.axis_index("c")

    @pl.when(wid < 16)
    def _():
        b = wid
        pltpu.sync_copy(idx_ref.at[b], idx_v)
        pltpu.async_copy(tab_ref.at[idx_v], rows_v, sem).wait()
        o0 = pl.multiple_of(32 * b, 32)
        handles = []
        for k in range(TOP_K):
            handles.append(pltpu.async_copy(
                rows_v.at[k, pl.ds(0, HALF)],
                key_ref.at[pl.ds(o0 + HALF * k, HALF)], sem))
            handles.append(pltpu.async_copy(
                rows_v.at[k, pl.ds(HALF, HALF)],
                val_ref.at[pl.ds(o0 + HALF * k, HALF)], sem))
        for h in handles:
            h.wait()

    @pl.when(wid >= 16)
    def _():
        w = wid - 16
        pltpu.sync_copy(idx_ref.at[w], pidx_v)
        pltpu.async_copy(pk_ref.at[pidx_v], pkrows_v, sem).wait()
        pltpu.sync_copy(pkrows_v, pkraw_ref.at[pl.ds(pl.multiple_of(8 * w, 8), 8)])


@functools.cache
def _sc_gather_fn():
    mesh = plsc.VectorSubcoreMesh(core_axis_name="c", subcore_axis_name="s")
    return pl.kernel(
        _sc_gather_body,
        out_type=(
            jax.ShapeDtypeStruct((B * TOP_K * HALF, D), jnp.float32),
            jax.ShapeDtypeStruct((B * TOP_K * HALF, D), jnp.float32),
            jax.ShapeDtypeStruct((NIDX, D), jnp.float32),
        ),
        mesh=mesh,
        scratch_types=(
            pltpu.VMEM((8,), jnp.int32),
            pltpu.VMEM((8, L2, D), jnp.float32),
            pltpu.VMEM((8,), jnp.int32),
            pltpu.VMEM((8, D), jnp.float32),
            pltpu.SemaphoreType.DMA,
        ),
    )


def kernel(x_embed, prompt_mask, cls_features, prompt, prompt_key):
    del cls_features
    key_rows, val_rows, pkraw = _sc_gather_fn()(prompt, prompt_mask, prompt_key)

    x_norm = _mean_norm(x_embed)
    pk_norm, similarity, bkn, rs = _pk_sim(x_norm, prompt_key, pkraw)

    key_prompt = key_rows.reshape(B, TOP_K * HALF, D)
    value_prompt = val_rows.reshape(B, TOP_K * HALF, D)
    batched_key_norm = bkn.reshape(B, TOP_K, D)
    reduce_sim = rs[0, 0]
    return (similarity, prompt_mask, key_prompt, value_prompt,
            batched_key_norm, pk_norm, x_norm, reduce_sim)


# speedup vs baseline: 2.3329x; 1.0347x over previous
import functools

import jax
import jax.numpy as jnp
from jax import lax
from jax.experimental import pallas as pl
from jax.experimental.pallas import tpu as pltpu
from jax.experimental.pallas import tpu_sc as plsc

SEQ, B, D = 2048, 16, 768
POOL, L2, TOP_K = 4096, 8, 8
HALF = L2 // 2
ROW = HALF * D
NIDX = B * TOP_K
EPS = 1e-12

_SEQ_BLK = 128


def _mean_norm_body(x_ref, o_ref):
    i = pl.program_id(0)
    part = jnp.sum(x_ref[...], axis=0)

    @pl.when(i == 0)
    def _():
        o_ref[...] = part

    @pl.when(i > 0)
    def _():
        o_ref[...] += part

    @pl.when(i == pl.num_programs(0) - 1)
    def _():
        m = o_ref[...] * (1.0 / SEQ)
        ssq = jnp.sum(m * m, axis=1, keepdims=True)
        o_ref[...] = m * lax.rsqrt(jnp.maximum(ssq, EPS))


def _mean_norm(x3d):
    return pl.pallas_call(
        _mean_norm_body,
        grid=(SEQ // _SEQ_BLK,),
        in_specs=[pl.BlockSpec((_SEQ_BLK, B, D), lambda i: (i, 0, 0))],
        out_specs=pl.BlockSpec((B, D), lambda i: (0, 0)),
        out_shape=jax.ShapeDtypeStruct((B, D), jnp.float32),
        compiler_params=pltpu.CompilerParams(
            dimension_semantics=("arbitrary",)),
    )(x3d)


_POOL_BLK = 2048


def _pk_sim_body(xn_ref, pk_ref, pkraw_ref, pkn_ref, sim_ref, bkn_ref, rs_ref):
    i = pl.program_id(0)
    pk = pk_ref[...]
    ssq = jnp.sum(pk * pk, axis=1, keepdims=True)
    pkn = pk * lax.rsqrt(jnp.maximum(ssq, EPS))
    pkn_ref[...] = pkn
    xn = xn_ref[...]
    sim_ref[...] = lax.dot_general(
        xn, pkn, (((1,), (1,)), ((), ())),
        preferred_element_type=jnp.float32)

    @pl.when(i == pl.num_programs(0) - 1)
    def _():
        g = pkraw_ref[...]
        gssq = jnp.sum(g * g, axis=1, keepdims=True)
        bkn = g * lax.rsqrt(jnp.maximum(gssq, EPS))
        bkn_ref[...] = bkn
        xr = jnp.broadcast_to(xn[:, None, :], (B, TOP_K, D)).reshape(NIDX, D)
        rs_ref[0, 0] = jnp.sum(bkn * xr) * (1.0 / (B * TOP_K))


def _pk_sim(x_norm, prompt_key, pkraw):
    return pl.pallas_call(
        _pk_sim_body,
        grid=(POOL // _POOL_BLK,),
        in_specs=[
            pl.BlockSpec((B, D), lambda i: (0, 0)),
            pl.BlockSpec((_POOL_BLK, D), lambda i: (i, 0)),
            pl.BlockSpec((NIDX, D), lambda i: (0, 0)),
        ],
        out_specs=[
            pl.BlockSpec((_POOL_BLK, D), lambda i: (i, 0)),
            pl.BlockSpec((B, _POOL_BLK), lambda i: (0, i)),
            pl.BlockSpec((NIDX, D), lambda i: (0, 0)),
            pl.BlockSpec(memory_space=pltpu.SMEM),
        ],
        out_shape=[
            jax.ShapeDtypeStruct((POOL, D), jnp.float32),
            jax.ShapeDtypeStruct((B, POOL), jnp.float32),
            jax.ShapeDtypeStruct((NIDX, D), jnp.float32),
            jax.ShapeDtypeStruct((1, 1), jnp.float32),
        ],
    )(x_norm, prompt_key, pkraw)


_LT = D // 128


def _sc_gather_body(tab_ref, idx_ref, pk_ref, key_ref, val_ref, pkraw_ref,
                    idx_v, rows_v, pidx_v, pkrows_v, sem):
    wid = lax.axis_index("s") * 2 + lax.axis_index("c")

    @pl.when(wid < 16)
    def _():
        b = wid
        pltpu.sync_copy(idx_ref.at[b], idx_v)
        pltpu.async_copy(tab_ref.at[idx_v], rows_v, sem).wait()
        o0 = pl.multiple_of(32 * b, 32)
        handles = []
        for k in range(TOP_K):
            handles.append(pltpu.async_copy(
                rows_v.at[k, pl.ds(0, HALF)],
                key_ref.at[pl.ds(o0 + HALF * k, HALF)], sem))
            handles.append(pltpu.async_copy(
                rows_v.at[k, pl.ds(HALF, HALF)],
                val_ref.at[pl.ds(o0 + HALF * k, HALF)], sem))
        for h in handles:
            h.wait()

    @pl.when(wid >= 16)
    def _():
        w = wid - 16
        pltpu.sync_copy(idx_ref.at[w], pidx_v)
        pltpu.async_copy(pk_ref.at[pidx_v], pkrows_v, sem).wait()
        pltpu.sync_copy(pkrows_v, pkraw_ref.at[pl.ds(pl.multiple_of(8 * w, 8), 8)])


@functools.cache
def _sc_gather_fn():
    mesh = plsc.VectorSubcoreMesh(core_axis_name="c", subcore_axis_name="s")
    return pl.kernel(
        _sc_gather_body,
        out_type=(
            jax.ShapeDtypeStruct((B * TOP_K * HALF, D), jnp.float32),
            jax.ShapeDtypeStruct((B * TOP_K * HALF, D), jnp.float32),
            jax.ShapeDtypeStruct((NIDX, D), jnp.float32),
        ),
        mesh=mesh,
        scratch_types=(
            pltpu.VMEM((8,), jnp.int32),
            pltpu.VMEM((8, L2, D), jnp.float32),
            pltpu.VMEM((8,), jnp.int32),
            pltpu.VMEM((8, D), jnp.float32),
            pltpu.SemaphoreType.DMA,
        ),
    )


def kernel(x_embed, prompt_mask, cls_features, prompt, prompt_key):
    del cls_features
    key_rows, val_rows, pkraw = _sc_gather_fn()(prompt, prompt_mask, prompt_key)

    x_norm = _mean_norm(x_embed)
    pk_norm, similarity, bkn, rs = _pk_sim(x_norm, prompt_key, pkraw)

    key_prompt = key_rows.reshape(B, TOP_K * HALF, D)
    value_prompt = val_rows.reshape(B, TOP_K * HALF, D)
    batched_key_norm = bkn.reshape(B, TOP_K, D)
    reduce_sim = rs[0, 0]
    return (similarity, prompt_mask, key_prompt, value_prompt,
            batched_key_norm, pk_norm, x_norm, reduce_sim)


# speedup vs baseline: 2.3972x; 1.0276x over previous
import functools

import jax
import jax.numpy as jnp
from jax import lax
from jax.experimental import pallas as pl
from jax.experimental.pallas import tpu as pltpu
from jax.experimental.pallas import tpu_sc as plsc

SEQ, B, D = 2048, 16, 768
POOL, L2, TOP_K = 4096, 8, 8
HALF = L2 // 2
ROW = HALF * D
NIDX = B * TOP_K
EPS = 1e-12

_SEQ_BLK = 128


def _mean_norm_body(x_ref, o_ref):
    i = pl.program_id(0)
    part = jnp.sum(x_ref[...], axis=0)

    @pl.when(i == 0)
    def _():
        o_ref[...] = part

    @pl.when(i > 0)
    def _():
        o_ref[...] += part

    @pl.when(i == pl.num_programs(0) - 1)
    def _():
        m = o_ref[...] * (1.0 / SEQ)
        ssq = jnp.sum(m * m, axis=1, keepdims=True)
        o_ref[...] = m * lax.rsqrt(jnp.maximum(ssq, EPS))


def _mean_norm(x3d):
    return pl.pallas_call(
        _mean_norm_body,
        grid=(SEQ // _SEQ_BLK,),
        in_specs=[pl.BlockSpec((_SEQ_BLK, B, D), lambda i: (i, 0, 0))],
        out_specs=pl.BlockSpec((B, D), lambda i: (0, 0)),
        out_shape=jax.ShapeDtypeStruct((B, D), jnp.float32),
        compiler_params=pltpu.CompilerParams(
            dimension_semantics=("arbitrary",)),
    )(x3d)


_POOL_BLK = 2048


def _pk_sim_body(xn_ref, pk_ref, pkraw_ref, pkn_ref, sim_ref, bkn_ref, rs_ref):
    i = pl.program_id(0)
    pk = pk_ref[...]
    ssq = jnp.sum(pk * pk, axis=1, keepdims=True)
    pkn = pk * lax.rsqrt(jnp.maximum(ssq, EPS))
    pkn_ref[...] = pkn
    xn = xn_ref[...]
    sim_ref[...] = lax.dot_general(
        xn, pkn, (((1,), (1,)), ((), ())),
        preferred_element_type=jnp.float32)

    @pl.when(i == pl.num_programs(0) - 1)
    def _():
        g = pkraw_ref[...]
        gssq = jnp.sum(g * g, axis=1, keepdims=True)
        bkn = g * lax.rsqrt(jnp.maximum(gssq, EPS))
        bkn_ref[...] = bkn
        xr = jnp.broadcast_to(xn[:, None, :], (B, TOP_K, D)).reshape(NIDX, D)
        rs_ref[0, 0] = jnp.sum(bkn * xr) * (1.0 / (B * TOP_K))


def _pk_sim(x_norm, prompt_key, pkraw):
    return pl.pallas_call(
        _pk_sim_body,
        grid=(POOL // _POOL_BLK,),
        in_specs=[
            pl.BlockSpec((B, D), lambda i: (0, 0)),
            pl.BlockSpec((_POOL_BLK, D), lambda i: (i, 0)),
            pl.BlockSpec((NIDX, D), lambda i: (0, 0)),
        ],
        out_specs=[
            pl.BlockSpec((_POOL_BLK, D), lambda i: (i, 0)),
            pl.BlockSpec((B, _POOL_BLK), lambda i: (0, i)),
            pl.BlockSpec((NIDX, D), lambda i: (0, 0)),
            pl.BlockSpec(memory_space=pltpu.SMEM),
        ],
        out_shape=[
            jax.ShapeDtypeStruct((POOL, D), jnp.float32),
            jax.ShapeDtypeStruct((B, POOL), jnp.float32),
            jax.ShapeDtypeStruct((NIDX, D), jnp.float32),
            jax.ShapeDtypeStruct((1, 1), jnp.float32),
        ],
    )(x_norm, prompt_key, pkraw)


_LT = D // 128


def _sc_gather_body(tab_ref, idx_ref, pk_ref, key_ref, val_ref, pkraw_ref,
                    idx_v, rows_v, pkrows_v, sem):
    b = lax.axis_index("s")

    pltpu.sync_copy(idx_ref.at[b], idx_v)
    pltpu.async_copy(tab_ref.at[idx_v], rows_v, sem).wait()
    o0 = pl.multiple_of(32 * b, 32)
    handles = []
    for k in range(TOP_K):
        handles.append(pltpu.async_copy(
            rows_v.at[k, pl.ds(0, HALF)],
            key_ref.at[pl.ds(o0 + HALF * k, HALF)], sem))
        handles.append(pltpu.async_copy(
            rows_v.at[k, pl.ds(HALF, HALF)],
            val_ref.at[pl.ds(o0 + HALF * k, HALF)], sem))
    handles.append(pltpu.async_copy(pk_ref.at[idx_v], pkrows_v, sem))
    for h in handles:
        h.wait()
    pltpu.sync_copy(pkrows_v, pkraw_ref.at[pl.ds(pl.multiple_of(8 * b, 8), 8)])


@functools.cache
def _sc_gather_fn():
    mesh = plsc.VectorSubcoreMesh(
        core_axis_name="c", subcore_axis_name="s", num_cores=1)
    return pl.kernel(
        _sc_gather_body,
        out_type=(
            jax.ShapeDtypeStruct((B * TOP_K * HALF, D), jnp.float32),
            jax.ShapeDtypeStruct((B * TOP_K * HALF, D), jnp.float32),
            jax.ShapeDtypeStruct((NIDX, D), jnp.float32),
        ),
        mesh=mesh,
        scratch_types=(
            pltpu.VMEM((8,), jnp.int32),
            pltpu.VMEM((8, L2, D), jnp.float32),
            pltpu.VMEM((8, D), jnp.float32),
            pltpu.SemaphoreType.DMA,
        ),
    )


def kernel(x_embed, prompt_mask, cls_features, prompt, prompt_key):
    del cls_features
    key_rows, val_rows, pkraw = _sc_gather_fn()(prompt, prompt_mask, prompt_key)

    x_norm = _mean_norm(x_embed)
    pk_norm, similarity, bkn, rs = _pk_sim(x_norm, prompt_key, pkraw)

    key_prompt = key_rows.reshape(B, TOP_K * HALF, D)
    value_prompt = val_rows.reshape(B, TOP_K * HALF, D)
    batched_key_norm = bkn.reshape(B, TOP_K, D)
    reduce_sim = rs[0, 0]
    return (similarity, prompt_mask, key_prompt, value_prompt,
            batched_key_norm, pk_norm, x_norm, reduce_sim)
